# baseline (speedup 1.0000x reference)
import jax
import jax.numpy as jnp
from jax import lax
from jax.experimental import pallas as pl
from jax.experimental.pallas import tpu as pltpu
from jax.experimental.pallas import tpu_sc as plsc

N = 100000
HIDDEN = 256
NSEG = 1024
OUT = HIDDEN * 3

L = 16
G = HIDDEN // L
NW = 32
SEG_PER_W = NSEG // NW
C = 64
NBUF = 4
WIN = 128
NPAD = 100096
SUB = NPAD // WIN
SUB_PAD = 896

_NEG = -3.0e38


def _vext(vec_ref, idx):
    base = lax.div(idx, 8) * 8
    vec = vec_ref[pl.ds(base, 16)]
    lane = idx - base
    sel = jnp.where(lax.iota(jnp.int32, 16) == lane, vec, -1)
    return jnp.max(sel)


def _acc_init():
    return tuple(
        jnp.zeros((L,), jnp.float32) if k % 2 == 0 else jnp.full((L,), _NEG)
        for k in range(2 * G)
    )


def _pool_body(h_hbm, sub_hbm, rows_hbm, x_hbm, sub_v, res_v, win_v,
               b0, b1, b2, b3, xstage, wsem, s0, s1, s2, s3):
    wid = lax.axis_index("s") * 2 + lax.axis_index("c")
    seg_base = wid * SEG_PER_W

    pltpu.sync_copy(sub_hbm, sub_v)
    lanes = lax.iota(jnp.int32, L)
    for grp in range(3):
        q = seg_base + grp * L + lanes

        lo = jnp.zeros((L,), jnp.int32)
        hi = jnp.full((L,), SUB, jnp.int32)
        for _ in range(10):
            mid = lax.div(lo + hi, 2)
            v = plsc.load_gather(sub_v, [mid])
            pred = v < q
            lo = jnp.where(pred, mid + 1, lo)
            hi = jnp.where(pred, hi, mid)
        c = lo

        row = jnp.maximum(c - 1, 0)
        pltpu.async_copy(rows_hbm.at[row], win_v, wsem).wait()
        flo = jnp.zeros((L,), jnp.int32)
        fhi = jnp.full((L,), WIN, jnp.int32)
        for _ in range(8):
            mid = jnp.minimum(lax.div(flo + fhi, 2), WIN - 1)
            v = plsc.load_gather(win_v, [lanes, mid])
            pred = v < q
            flo = jnp.where(pred, mid + 1, flo)
            fhi = jnp.where(pred, fhi, mid)
        off = jnp.where(c == 0, 0, (c - 1) * WIN + flo)
        res_v[pl.ds(grp * L, L)] = off

    row_lo = _vext(res_v, 0)
    row_hi = _vext(res_v, SEG_PER_W)

    a0 = lax.div(row_lo, 8) * 8
    nch = jnp.maximum(lax.div(row_hi - a0 + (C - 1), C), 1)

    bufs = (b0, b1, b2, b3)
    sems = (s0, s1, s2, s3)

    def issue(i, k):
        bc = jnp.minimum(a0 + i * C, N - C)
        pltpu.async_copy(h_hbm.at[pl.ds(bc, C)], bufs[k], sems[k])

    issue(0, 0)
    for k in range(1, NBUF):
        @pl.when(k < nch)
        def _(k=k):
            issue(k, k)

    def process_chunk(st, buf, bc, chunk_end):
        def cond_fn(st):
            sl, end_cur, row = st[-4], st[-2], st[-1]
            return (row < chunk_end) | (
                (end_cur <= chunk_end) & (sl < SEG_PER_W))

        def body_fn(st):
            accs = st[:-4]
            sl, seg_start, end_cur, row = st[-4:]
            e = jnp.minimum(end_cur, chunk_end)
            n8 = lax.div(e - row, 8)

            def grp8(gi, accs):
                base = row - bc + gi * 8
                new = list(accs)
                for r in range(8):
                    for g in range(G):
                        v = buf[base + r, pl.ds(g * L, L)]
                        new[2 * g] = new[2 * g] + v
                        new[2 * g + 1] = jnp.maximum(new[2 * g + 1], v)
                return tuple(new)

            accs = lax.fori_loop(0, n8, grp8, accs)

            def acc_row(i, accs):
                j = i - bc
                new = list(accs)
                for g in range(G):
                    v = buf[j, pl.ds(g * L, L)]
                    new[2 * g] = new[2 * g] + v
                    new[2 * g + 1] = jnp.maximum(new[2 * g + 1], v)
                return tuple(new)

            accs = lax.fori_loop(row + n8 * 8, e, acc_row, accs)

            def fin(ops):
                accs = ops[:-3]
                sl, seg_start, end_cur = ops[-3:]
                cnt = end_cur - seg_start
                cntf = jnp.full((L,), 1.0, jnp.float32) * cnt.astype(jnp.float32)
                inv = 1.0 / jnp.maximum(cntf, 1.0)
                nonempty = cnt > 0
                for g in range(G):
                    sacc = accs[2 * g]
                    macc = jnp.where(nonempty, accs[2 * g + 1], 0.0)
                    xstage[sl, pl.ds(g * L, L)] = sacc * inv
                    xstage[sl, pl.ds(HIDDEN + g * L, L)] = macc
                    xstage[sl, pl.ds(2 * HIDDEN + g * L, L)] = sacc
                return _acc_init() + (sl + 1, end_cur, _vext(res_v, sl + 2))

            def keep(ops):
                return ops

            out = lax.cond(end_cur <= chunk_end, fin, keep,
                           accs + (sl, seg_start, end_cur))
            return out + (e,)

        return lax.while_loop(cond_fn, body_fn, st)

    def outer(ii, st):
        for k in range(NBUF):
            i = ii * NBUF + k

            def do(st, i=i, k=k):
                pltpu.make_async_copy(h_hbm.at[pl.ds(0, C)], bufs[k],
                                      sems[k]).wait()
                b = a0 + i * C
                bc = jnp.minimum(b, N - C)
                chunk_end = jnp.minimum(b + C, row_hi)
                st = process_chunk(st, bufs[k], bc, chunk_end)

                @pl.when(i + NBUF < nch)
                def _():
                    issue(i + NBUF, k)
                return st

            st = lax.cond(i < nch, do, lambda st: st, st)
        return st

    init_st = _acc_init() + (
        jnp.int32(0), row_lo, _vext(res_v, 1), row_lo)
    nouter = lax.div(nch + (NBUF - 1), NBUF)
    lax.fori_loop(0, nouter, outer, init_st)
    pltpu.sync_copy(xstage, x_hbm.at[pl.ds(seg_base, SEG_PER_W)])


_pool = pl.kernel(
    _pool_body,
    out_type=jax.ShapeDtypeStruct((NSEG, OUT), jnp.float32),
    mesh=plsc.VectorSubcoreMesh(core_axis_name="c", subcore_axis_name="s"),
    compiler_params=pltpu.CompilerParams(needs_layout_passes=False),
    scratch_types=[
        pltpu.VMEM((SUB_PAD,), jnp.int32),
        pltpu.VMEM((64,), jnp.int32),
        pltpu.VMEM((L, WIN), jnp.int32),
        pltpu.VMEM((C, HIDDEN), jnp.float32),
        pltpu.VMEM((C, HIDDEN), jnp.float32),
        pltpu.VMEM((C, HIDDEN), jnp.float32),
        pltpu.VMEM((C, HIDDEN), jnp.float32),
        pltpu.VMEM((SEG_PER_W, OUT), jnp.float32),
        pltpu.SemaphoreType.DMA,
        pltpu.SemaphoreType.DMA,
        pltpu.SemaphoreType.DMA,
        pltpu.SemaphoreType.DMA,
        pltpu.SemaphoreType.DMA,
    ],
)


def _mlp_body(x_ref, g_ref, bta_ref, w_ref, b_ref, o_ref):
    x = x_ref[...]
    mu = jnp.mean(x, axis=-1, keepdims=True)
    xc = x - mu
    var = jnp.mean(xc * xc, axis=-1, keepdims=True)
    xn = xc * lax.rsqrt(var + 1e-5)
    xn = xn * g_ref[...] + bta_ref[...]
    y = lax.dot_general(xn, w_ref[...], (((1,), (1,)), ((), ())),
                        preferred_element_type=jnp.float32)
    y = y + b_ref[...]
    o_ref[...] = y * 0.5 * (1.0 + lax.erf(y * 0.7071067811865476))


def _mlp(x, ln_gamma, ln_beta, W, b):
    TM = 1024
    return pl.pallas_call(
        _mlp_body,
        out_shape=jax.ShapeDtypeStruct((NSEG, OUT), jnp.float32),
        grid=(NSEG // TM,),
        in_specs=[
            pl.BlockSpec((TM, OUT), lambda i: (i, 0)),
            pl.BlockSpec((1, OUT), lambda i: (0, 0)),
            pl.BlockSpec((1, OUT), lambda i: (0, 0)),
            pl.BlockSpec((OUT, OUT), lambda i: (0, 0)),
            pl.BlockSpec((1, OUT), lambda i: (0, 0)),
        ],
        out_specs=pl.BlockSpec((TM, OUT), lambda i: (i, 0)),
    )(x, ln_gamma, ln_beta, W, b)


def kernel(h, batch_vec, ln_gamma, ln_beta, W, b):
    seg = batch_vec.astype(jnp.int32)
    segp = jnp.concatenate([seg, jnp.full((NPAD - N,), jnp.int32(1 << 30))])
    sub = jnp.concatenate(
        [segp[::WIN], jnp.full((SUB_PAD - SUB,), jnp.int32(1 << 30))])
    rows = segp.reshape(SUB, WIN)
    x = _pool(h, sub, rows)
    return _mlp(x, ln_gamma.reshape(1, OUT), ln_beta.reshape(1, OUT), W,
                b.reshape(1, OUT))


# speedup vs baseline: 1.4740x; 1.4740x over previous
import jax
import jax.numpy as jnp
from jax import lax
from jax.experimental import pallas as pl
from jax.experimental.pallas import tpu as pltpu
from jax.experimental.pallas import tpu_sc as plsc

N = 100000
HIDDEN = 256
NSEG = 1024
OUT = HIDDEN * 3

L = 16
G = HIDDEN // L
NW = 32
SEG_PER_W = NSEG // NW
C = 64
NBUF = 4
WIN = 128
NPAD = 100096
SUB = NPAD // WIN
SUB_PAD = 896

_NEG = -3.0e38


def _vext(vec_ref, idx):
    base = lax.div(idx, 8) * 8
    vec = vec_ref[pl.ds(base, 16)]
    lane = idx - base
    sel = jnp.where(lax.iota(jnp.int32, 16) == lane, vec, -1)
    return jnp.max(sel)


def _acc_init():
    return tuple(
        jnp.zeros((L,), jnp.float32) if k % 2 == 0 else jnp.full((L,), _NEG)
        for k in range(2 * G)
    )


def _pool_body(h_hbm, sub_hbm, rows_hbm, x_hbm, sub_v, res_v, win_v,
               b0, b1, b2, b3, xstage, wsem, s0, s1, s2, s3):
    wid = lax.axis_index("s") * 2 + lax.axis_index("c")
    seg_base = wid * SEG_PER_W

    pltpu.sync_copy(sub_hbm, sub_v)
    lanes = lax.iota(jnp.int32, L)
    for grp in range(3):
        q = seg_base + grp * L + lanes

        lo = jnp.zeros((L,), jnp.int32)
        hi = jnp.full((L,), SUB, jnp.int32)
        for _ in range(10):
            mid = lax.div(lo + hi, 2)
            v = plsc.load_gather(sub_v, [mid])
            pred = v < q
            lo = jnp.where(pred, mid + 1, lo)
            hi = jnp.where(pred, hi, mid)
        c = lo

        row = jnp.maximum(c - 1, 0)
        pltpu.async_copy(rows_hbm.at[row], win_v, wsem).wait()
        flo = jnp.zeros((L,), jnp.int32)
        fhi = jnp.full((L,), WIN, jnp.int32)
        for _ in range(8):
            mid = jnp.minimum(lax.div(flo + fhi, 2), WIN - 1)
            v = plsc.load_gather(win_v, [lanes, mid])
            pred = v < q
            flo = jnp.where(pred, mid + 1, flo)
            fhi = jnp.where(pred, fhi, mid)
        off = jnp.where(c == 0, 0, (c - 1) * WIN + flo)
        res_v[pl.ds(grp * L, L)] = off

    row_lo = _vext(res_v, 0)
    row_hi = _vext(res_v, SEG_PER_W)

    a0 = lax.div(row_lo, 8) * 8
    nch = jnp.maximum(lax.div(row_hi - a0 + (C - 1), C), 1)

    bufs = (b0, b1, b2, b3)
    sems = (s0, s1, s2, s3)

    def issue(i, k):
        bc = jnp.minimum(a0 + i * C, N - C)
        pltpu.async_copy(h_hbm.at[pl.ds(bc, C)], bufs[k], sems[k])

    issue(0, 0)
    for k in range(1, NBUF):
        @pl.when(k < nch)
        def _(k=k):
            issue(k, k)

    def process_chunk(st, buf, bc, chunk_end):
        def cond_fn(st):
            sl, end_cur, row = st[-4], st[-2], st[-1]
            return (row < chunk_end) | (
                (end_cur <= chunk_end) & (sl < SEG_PER_W))

        def body_fn(st):
            accs = st[:-4]
            sl, seg_start, end_cur, row = st[-4:]
            e = jnp.minimum(end_cur, chunk_end)

            def acc_row(i, accs):
                j = i - bc
                new = list(accs)
                for g in range(G):
                    v = buf[j, pl.ds(g * L, L)]
                    new[2 * g] = new[2 * g] + v
                    new[2 * g + 1] = jnp.maximum(new[2 * g + 1], v)
                return tuple(new)

            accs = lax.fori_loop(row, e, acc_row, accs)

            def fin(ops):
                accs = ops[:-3]
                sl, seg_start, end_cur = ops[-3:]
                cnt = end_cur - seg_start
                cntf = jnp.full((L,), 1.0, jnp.float32) * cnt.astype(jnp.float32)
                inv = 1.0 / jnp.maximum(cntf, 1.0)
                nonempty = cnt > 0
                for g in range(G):
                    sacc = accs[2 * g]
                    macc = jnp.where(nonempty, accs[2 * g + 1], 0.0)
                    xstage[sl, pl.ds(g * L, L)] = sacc * inv
                    xstage[sl, pl.ds(HIDDEN + g * L, L)] = macc
                    xstage[sl, pl.ds(2 * HIDDEN + g * L, L)] = sacc
                return _acc_init() + (sl + 1, end_cur, _vext(res_v, sl + 2))

            def keep(ops):
                return ops

            out = lax.cond(end_cur <= chunk_end, fin, keep,
                           accs + (sl, seg_start, end_cur))
            return out + (e,)

        return lax.while_loop(cond_fn, body_fn, st)

    def outer(ii, st):
        for k in range(NBUF):
            i = ii * NBUF + k

            def do(st, i=i, k=k):
                pltpu.make_async_copy(h_hbm.at[pl.ds(0, C)], bufs[k],
                                      sems[k]).wait()
                b = a0 + i * C
                bc = jnp.minimum(b, N - C)
                chunk_end = jnp.minimum(b + C, row_hi)
                st = process_chunk(st, bufs[k], bc, chunk_end)

                @pl.when(i + NBUF < nch)
                def _():
                    issue(i + NBUF, k)
                return st

            st = lax.cond(i < nch, do, lambda st: st, st)
        return st

    init_st = _acc_init() + (
        jnp.int32(0), row_lo, _vext(res_v, 1), row_lo)
    nouter = lax.div(nch + (NBUF - 1), NBUF)
    lax.fori_loop(0, nouter, outer, init_st)
    pltpu.sync_copy(xstage, x_hbm.at[pl.ds(seg_base, SEG_PER_W)])


_pool = pl.kernel(
    _pool_body,
    out_type=jax.ShapeDtypeStruct((NSEG, OUT), jnp.float32),
    mesh=plsc.VectorSubcoreMesh(core_axis_name="c", subcore_axis_name="s"),
    compiler_params=pltpu.CompilerParams(needs_layout_passes=False),
    scratch_types=[
        pltpu.VMEM((SUB_PAD,), jnp.int32),
        pltpu.VMEM((64,), jnp.int32),
        pltpu.VMEM((L, WIN), jnp.int32),
        pltpu.VMEM((C, HIDDEN), jnp.float32),
        pltpu.VMEM((C, HIDDEN), jnp.float32),
        pltpu.VMEM((C, HIDDEN), jnp.float32),
        pltpu.VMEM((C, HIDDEN), jnp.float32),
        pltpu.VMEM((SEG_PER_W, OUT), jnp.float32),
        pltpu.SemaphoreType.DMA,
        pltpu.SemaphoreType.DMA,
        pltpu.SemaphoreType.DMA,
        pltpu.SemaphoreType.DMA,
        pltpu.SemaphoreType.DMA,
    ],
)


def _mlp_body(x_ref, g_ref, bta_ref, w_ref, b_ref, o_ref):
    x = x_ref[...]
    mu = jnp.mean(x, axis=-1, keepdims=True)
    xc = x - mu
    var = jnp.mean(xc * xc, axis=-1, keepdims=True)
    xn = xc * lax.rsqrt(var + 1e-5)
    xn = xn * g_ref[...] + bta_ref[...]
    y = lax.dot_general(xn, w_ref[...], (((1,), (1,)), ((), ())),
                        preferred_element_type=jnp.float32)
    y = y + b_ref[...]
    o_ref[...] = y * 0.5 * (1.0 + lax.erf(y * 0.7071067811865476))


def _mlp(x, ln_gamma, ln_beta, W, b):
    TM = 1024
    return pl.pallas_call(
        _mlp_body,
        out_shape=jax.ShapeDtypeStruct((NSEG, OUT), jnp.float32),
        grid=(NSEG // TM,),
        in_specs=[
            pl.BlockSpec((TM, OUT), lambda i: (i, 0)),
            pl.BlockSpec((1, OUT), lambda i: (0, 0)),
            pl.BlockSpec((1, OUT), lambda i: (0, 0)),
            pl.BlockSpec((OUT, OUT), lambda i: (0, 0)),
            pl.BlockSpec((1, OUT), lambda i: (0, 0)),
        ],
        out_specs=pl.BlockSpec((TM, OUT), lambda i: (i, 0)),
    )(x, ln_gamma, ln_beta, W, b)


def kernel(h, batch_vec, ln_gamma, ln_beta, W, b):
    seg = batch_vec.astype(jnp.int32)
    segp = jnp.concatenate([seg, jnp.full((NPAD - N,), jnp.int32(1 << 30))])
    sub = jnp.concatenate(
        [segp[::WIN], jnp.full((SUB_PAD - SUB,), jnp.int32(1 << 30))])
    rows = segp.reshape(SUB, WIN)
    x = _pool(h, sub, rows)
    return _mlp(x, ln_gamma.reshape(1, OUT), ln_beta.reshape(1, OUT), W,
                b.reshape(1, OUT))


# speedup vs baseline: 1.5256x; 1.0350x over previous
import jax
import jax.numpy as jnp
from jax import lax
from jax.experimental import pallas as pl
from jax.experimental.pallas import tpu as pltpu
from jax.experimental.pallas import tpu_sc as plsc

N = 100000
HIDDEN = 256
NSEG = 1024
OUT = HIDDEN * 3

L = 16
G = HIDDEN // L
NW = 32
SEG_PER_W = NSEG // NW
C = 120
NBUF = 3
WIN = 128
NPAD = 100096
SUB = NPAD // WIN
SUB_PAD = 896

_NEG = -3.0e38


def _vext(vec_ref, idx):
    base = lax.div(idx, 8) * 8
    vec = vec_ref[pl.ds(base, 16)]
    lane = idx - base
    sel = jnp.where(lax.iota(jnp.int32, 16) == lane, vec, -1)
    return jnp.max(sel)


def _acc_init():
    return tuple(
        jnp.zeros((L,), jnp.float32) if k % 2 == 0 else jnp.full((L,), _NEG)
        for k in range(2 * G)
    )


def _pool_body(h_hbm, sub_hbm, rows_hbm, x_hbm, sub_v, res_v, win_v,
               b0, b1, b2, xstage, w0, w1, w2, s0, s1, s2):
    wid = lax.axis_index("s") * 2 + lax.axis_index("c")
    seg_base = wid * SEG_PER_W

    pltpu.sync_copy(sub_hbm, sub_v)
    lanes = lax.iota(jnp.int32, L)
    wsems = (w0, w1, w2)
    cs = []
    for grp in range(3):
        q = seg_base + grp * L + lanes

        lo = jnp.zeros((L,), jnp.int32)
        hi = jnp.full((L,), SUB, jnp.int32)
        for _ in range(10):
            mid = lax.div(lo + hi, 2)
            v = plsc.load_gather(sub_v, [mid])
            pred = v < q
            lo = jnp.where(pred, mid + 1, lo)
            hi = jnp.where(pred, hi, mid)
        c = lo
        cs.append(c)
        pltpu.async_copy(rows_hbm.at[jnp.maximum(c - 1, 0)],
                         win_v.at[grp], wsems[grp])

    for grp in range(3):
        q = seg_base + grp * L + lanes
        c = cs[grp]
        pltpu.make_async_copy(rows_hbm.at[jnp.maximum(c - 1, 0)],
                              win_v.at[grp], wsems[grp]).wait()
        flo = jnp.zeros((L,), jnp.int32)
        fhi = jnp.full((L,), WIN, jnp.int32)
        for _ in range(8):
            mid = jnp.minimum(lax.div(flo + fhi, 2), WIN - 1)
            v = plsc.load_gather(win_v, [jnp.full((L,), grp, jnp.int32),
                                         lanes, mid])
            pred = v < q
            flo = jnp.where(pred, mid + 1, flo)
            fhi = jnp.where(pred, fhi, mid)
        off = jnp.where(c == 0, 0, (c - 1) * WIN + flo)
        res_v[pl.ds(grp * L, L)] = off

    row_lo = _vext(res_v, 0)
    row_hi = _vext(res_v, SEG_PER_W)

    a0 = lax.div(row_lo, 8) * 8
    nch = jnp.maximum(lax.div(row_hi - a0 + (C - 1), C), 1)

    bufs = (b0, b1, b2)
    sems = (s0, s1, s2)

    def issue(i, k):
        bc = jnp.minimum(a0 + i * C, N - C)
        pltpu.async_copy(h_hbm.at[pl.ds(bc, C)], bufs[k], sems[k])

    issue(0, 0)
    for k in range(1, NBUF):
        @pl.when(k < nch)
        def _(k=k):
            issue(k, k)

    def process_chunk(st, buf, bc, chunk_end):
        def cond_fn(st):
            sl, end_cur, row = st[-4], st[-2], st[-1]
            return (row < chunk_end) | (
                (end_cur <= chunk_end) & (sl < SEG_PER_W))

        def body_fn(st):
            accs = st[:-4]
            sl, seg_start, end_cur, row = st[-4:]
            e = jnp.minimum(end_cur, chunk_end)

            def acc_row(i, accs):
                j = i - bc
                new = list(accs)
                for g in range(G):
                    v = buf[j, pl.ds(g * L, L)]
                    new[2 * g] = new[2 * g] + v
                    new[2 * g + 1] = jnp.maximum(new[2 * g + 1], v)
                return tuple(new)

            accs = lax.fori_loop(row, e, acc_row, accs)

            def fin(ops):
                accs = ops[:-3]
                sl, seg_start, end_cur = ops[-3:]
                cnt = end_cur - seg_start
                cntf = jnp.full((L,), 1.0, jnp.float32) * cnt.astype(jnp.float32)
                inv = 1.0 / jnp.maximum(cntf, 1.0)
                nonempty = cnt > 0
                for g in range(G):
                    sacc = accs[2 * g]
                    macc = jnp.where(nonempty, accs[2 * g + 1], 0.0)
                    xstage[sl, pl.ds(g * L, L)] = sacc * inv
                    xstage[sl, pl.ds(HIDDEN + g * L, L)] = macc
                    xstage[sl, pl.ds(2 * HIDDEN + g * L, L)] = sacc
                return _acc_init() + (sl + 1, end_cur, _vext(res_v, sl + 2))

            def keep(ops):
                return ops

            out = lax.cond(end_cur <= chunk_end, fin, keep,
                           accs + (sl, seg_start, end_cur))
            return out + (e,)

        return lax.while_loop(cond_fn, body_fn, st)

    def outer(ii, st):
        for k in range(NBUF):
            i = ii * NBUF + k

            def do(st, i=i, k=k):
                pltpu.make_async_copy(h_hbm.at[pl.ds(0, C)], bufs[k],
                                      sems[k]).wait()
                b = a0 + i * C
                bc = jnp.minimum(b, N - C)
                chunk_end = jnp.minimum(b + C, row_hi)
                st = process_chunk(st, bufs[k], bc, chunk_end)

                @pl.when(i + NBUF < nch)
                def _():
                    issue(i + NBUF, k)
                return st

            st = lax.cond(i < nch, do, lambda st: st, st)
        return st

    init_st = _acc_init() + (
        jnp.int32(0), row_lo, _vext(res_v, 1), row_lo)
    nouter = lax.div(nch + (NBUF - 1), NBUF)
    lax.fori_loop(0, nouter, outer, init_st)
    pltpu.sync_copy(xstage, x_hbm.at[pl.ds(seg_base, SEG_PER_W)])


_pool = pl.kernel(
    _pool_body,
    out_type=jax.ShapeDtypeStruct((NSEG, OUT), jnp.float32),
    mesh=plsc.VectorSubcoreMesh(core_axis_name="c", subcore_axis_name="s"),
    compiler_params=pltpu.CompilerParams(needs_layout_passes=False),
    scratch_types=[
        pltpu.VMEM((SUB_PAD,), jnp.int32),
        pltpu.VMEM((64,), jnp.int32),
        pltpu.VMEM((3, L, WIN), jnp.int32),
        pltpu.VMEM((C, HIDDEN), jnp.float32),
        pltpu.VMEM((C, HIDDEN), jnp.float32),
        pltpu.VMEM((C, HIDDEN), jnp.float32),
        pltpu.VMEM((SEG_PER_W, OUT), jnp.float32),
        pltpu.SemaphoreType.DMA,
        pltpu.SemaphoreType.DMA,
        pltpu.SemaphoreType.DMA,
        pltpu.SemaphoreType.DMA,
        pltpu.SemaphoreType.DMA,
        pltpu.SemaphoreType.DMA,
    ],
)


def _mlp_body(x_ref, g_ref, bta_ref, w_ref, b_ref, o_ref):
    x = x_ref[...]
    mu = jnp.mean(x, axis=-1, keepdims=True)
    xc = x - mu
    var = jnp.mean(xc * xc, axis=-1, keepdims=True)
    xn = xc * lax.rsqrt(var + 1e-5)
    xn = xn * g_ref[...] + bta_ref[...]
    y = lax.dot_general(xn, w_ref[...], (((1,), (1,)), ((), ())),
                        preferred_element_type=jnp.float32)
    y = y + b_ref[...]
    o_ref[...] = y * 0.5 * (1.0 + lax.erf(y * 0.7071067811865476))


def _mlp(x, ln_gamma, ln_beta, W, b):
    TM = 1024
    return pl.pallas_call(
        _mlp_body,
        out_shape=jax.ShapeDtypeStruct((NSEG, OUT), jnp.float32),
        grid=(NSEG // TM,),
        in_specs=[
            pl.BlockSpec((TM, OUT), lambda i: (i, 0)),
            pl.BlockSpec((1, OUT), lambda i: (0, 0)),
            pl.BlockSpec((1, OUT), lambda i: (0, 0)),
            pl.BlockSpec((OUT, OUT), lambda i: (0, 0)),
            pl.BlockSpec((1, OUT), lambda i: (0, 0)),
        ],
        out_specs=pl.BlockSpec((TM, OUT), lambda i: (i, 0)),
    )(x, ln_gamma, ln_beta, W, b)


def kernel(h, batch_vec, ln_gamma, ln_beta, W, b):
    seg = batch_vec.astype(jnp.int32)
    segp = jnp.concatenate([seg, jnp.full((NPAD - N,), jnp.int32(1 << 30))])
    sub = jnp.concatenate(
        [segp[::WIN], jnp.full((SUB_PAD - SUB,), jnp.int32(1 << 30))])
    rows = segp.reshape(SUB, WIN)
    x = _pool(h, sub, rows)
    return _mlp(x, ln_gamma.reshape(1, OUT), ln_beta.reshape(1, OUT), W,
                b.reshape(1, OUT))
